# Initial kernel scaffold; baseline (speedup 1.0000x reference)
import jax
import jax.numpy as jnp
from jax.experimental import pallas as pl


def kernel(x, edge_index, batch, embedding, gru_w_ih, gru_w_hh, W1, W2, b2, Wq, bq, Wt):
    raise NotImplementedError("write your pallas kernel here")


# speedup vs baseline: 4.5298x; 4.5298x over previous
import functools

import jax
import jax.numpy as jnp
from jax import lax
from jax.experimental import pallas as pl
from jax.experimental.pallas import tpu as pltpu
from jax.experimental.pallas import tpu_sc as plsc

N = 10000
E = 320000
H = 128
T = 512
G = 256

NC = 2
NS = 16
HH = H // NC
EPT = E // NS
CH = 80
NCH = EPT // CH
NP = 10240
RPT = NP // NS
ZR = 128

def _build_sc_msg():
    mesh = plsc.VectorSubcoreMesh(
        core_axis_name="c", subcore_axis_name="s",
        num_cores=NC, num_subcores=NS,
    )

    @functools.partial(
        pl.kernel,
        out_type=jax.ShapeDtypeStruct((NC, NP, HH), jnp.float32),
        mesh=mesh,
        scratch_types=[
            pltpu.VMEM((N,), jnp.int32),
            pltpu.VMEM((NCH, CH), jnp.int32),
            pltpu.VMEM((NCH, CH), jnp.int32),
            pltpu.VMEM((NCH, CH), jnp.int32),
            pltpu.VMEM((CH, HH), jnp.float32),
            pltpu.VMEM((ZR, HH), jnp.float32),
            pltpu.VMEM_SHARED((NP, HH), jnp.float32),
            pltpu.SemaphoreType.DMA,
        ],
        compiler_params=pltpu.CompilerParams(
            needs_layout_passes=False, use_tc_tiling_on_sc=False),
    )
    def sc_msg(xs_hbm, src_hbm, dst_hbm, emb_hbm, zeros_hbm, out_hbm,
               xs_v, src_v, dst_v, t_v, rows_v, zbuf, msg_sh, sem):
        c = lax.axis_index("c")
        s = lax.axis_index("s")

        pltpu.sync_copy(zeros_hbm, zbuf)
        for k in range(RPT // ZR):
            pltpu.sync_copy(zbuf, msg_sh.at[pl.ds(s * RPT + k * ZR, ZR)])
        plsc.subcore_barrier()

        pltpu.sync_copy(xs_hbm, xs_v)
        pltpu.sync_copy(src_hbm.at[s], src_v)
        pltpu.sync_copy(dst_hbm.at[s], dst_v)

        def t_body(r, _):
            for c5 in range(CH // 16):
                src16 = src_v[r, pl.ds(c5 * 16, 16)]
                t16 = plsc.load_gather(xs_v, [src16])
                t_v[r, pl.ds(c5 * 16, 16)] = t16
            return 0

        lax.fori_loop(0, NCH, t_body, 0)

        def e_body(j, _):
            pltpu.async_copy(emb_hbm.at[c].at[t_v.at[j]], rows_v, sem).wait()
            pltpu.sync_copy(rows_v, msg_sh.at[dst_v.at[j]], add=True)
            return 0

        lax.fori_loop(0, NCH, e_body, 0)
        plsc.subcore_barrier()

        for k in range(RPT // ZR):
            start = s * RPT + k * ZR
            pltpu.sync_copy(msg_sh.at[pl.ds(start, ZR)], zbuf)
            pltpu.sync_copy(zbuf, out_hbm.at[c].at[pl.ds(start, ZR)])

    return sc_msg


_SC_MSG_CACHE = []


def _sc_msg(*args):
    if not _SC_MSG_CACHE:
        _SC_MSG_CACHE.append(_build_sc_msg())
    return _SC_MSG_CACHE[0](*args)


BLK = 1000
NB = N // BLK
_F32 = jnp.float32
_PREC = lax.Precision.HIGHEST


def _dot_t(a, b):
    return lax.dot_general(a, b, (((1,), (1,)), ((), ())),
                           precision=_PREC, preferred_element_type=_F32)


def _dot(a, b):
    return lax.dot_general(a, b, (((1,), (0,)), ((), ())),
                           precision=_PREC, preferred_element_type=_F32)


def _dot_c0(a, b):
    return lax.dot_general(a, b, (((0,), (0,)), ((), ())),
                           precision=_PREC, preferred_element_type=_F32)


def _tc_body(msg_ref, xs_ref, b_ref, bn_ref, emb_ref, wih_ref, whh_ref,
             w1_ref, w2_ref, b2_ref, wq_ref, bq_ref, wt_ref, out_ref,
             h_sc, wl_sc, wg_sc):
    emb = emb_ref[...]
    wih = wih_ref[...]
    whh = whh_ref[...]
    wl_sc[...] = jnp.zeros((G, H), _F32)
    wg_sc[...] = jnp.zeros((G, H), _F32)

    def pass1(i, _):
        ds = pl.ds(i * BLK, BLK)
        xs = xs_ref[ds, :]
        ohe = (lax.broadcasted_iota(jnp.int32, (BLK, T), 1) == xs
               ).astype(_F32)
        embb = _dot(ohe, emb)
        msgb = jnp.concatenate(
            [msg_ref[0, ds, :], msg_ref[1, ds, :]], axis=1)
        gi = _dot_t(msgb, wih)
        gh = _dot_t(embb, whh)
        r = jax.nn.sigmoid(gi[:, :H] + gh[:, :H])
        z = jax.nn.sigmoid(gi[:, H:2 * H] + gh[:, H:2 * H])
        n = jnp.tanh(gi[:, 2 * H:] + r * gh[:, 2 * H:])
        hb = (1.0 - z) * n + z * embb
        h_sc[ds, :] = hb
        b = b_ref[ds, :]
        ohb = (lax.broadcasted_iota(jnp.int32, (BLK, G), 1) == b
               ).astype(_F32)
        lastm = (b != bn_ref[ds, :]).astype(_F32)
        wl_sc[...] += _dot_c0(ohb * lastm, hb)
        return 0

    lax.fori_loop(0, NB, pass1, 0)
    wl = wl_sc[...]

    def pass2(i, _):
        ds = pl.ds(i * BLK, BLK)
        hb = h_sc[ds, :]
        b = b_ref[ds, :]
        ohb = (lax.broadcasted_iota(jnp.int32, (BLK, G), 1) == b
               ).astype(_F32)
        wgr = _dot(ohb, wl)
        q1 = _dot_t(wgr, w1_ref[...])
        q2 = _dot_t(hb, w2_ref[...]) + b2_ref[...]
        alpha = _dot_t(jax.nn.sigmoid(q1 + q2), wq_ref[...]) + bq_ref[...]
        a = alpha * hb
        wg_sc[...] += _dot_c0(ohb, a)
        return 0

    lax.fori_loop(0, NB, pass2, 0)
    wcat = jnp.concatenate([wl, wg_sc[...]], axis=1)
    wv = _dot_t(wcat, wt_ref[...])
    out_ref[...] = _dot_t(wv, emb)


@jax.jit
def _run(xs, src, dst, batch, bnext, embedding, gru_w_ih, gru_w_hh,
         W1, W2, b2, Wq, bq, Wt):
    zeros = jnp.zeros((ZR, HH), jnp.float32)
    emb2 = jnp.stack([embedding[:, :HH], embedding[:, HH:]])
    msg2 = _sc_msg(xs, src, dst, emb2, zeros)
    return pl.pallas_call(
        _tc_body,
        out_shape=jax.ShapeDtypeStruct((G, T), jnp.float32),
        scratch_shapes=[
            pltpu.VMEM((N, H), jnp.float32),
            pltpu.VMEM((G, H), jnp.float32),
            pltpu.VMEM((G, H), jnp.float32),
        ],
    )(msg2, xs.reshape(N, 1), batch.reshape(N, 1), bnext.reshape(N, 1),
      embedding, gru_w_ih, gru_w_hh, W1, W2, b2.reshape(1, H), Wq,
      bq.reshape(1, H), Wt)


def kernel(x, edge_index, batch, embedding, gru_w_ih, gru_w_hh,
           W1, W2, b2, Wq, bq, Wt):
    xs = x[:, 0].astype(jnp.int32)
    src = edge_index[0].astype(jnp.int32).reshape(NS, NCH, CH)
    dst = edge_index[1].astype(jnp.int32).reshape(NS, NCH, CH)
    bnext = jnp.concatenate(
        [batch[1:], jnp.full((1,), G, batch.dtype)]).astype(jnp.int32)
    return _run(xs, src, dst, batch.astype(jnp.int32), bnext, embedding,
                gru_w_ih, gru_w_hh, W1, W2, b2, Wq, bq, Wt)


# speedup vs baseline: 5.1473x; 1.1363x over previous
import functools

import jax
import jax.numpy as jnp
from jax import lax
from jax.experimental import pallas as pl
from jax.experimental.pallas import tpu as pltpu
from jax.experimental.pallas import tpu_sc as plsc

N = 10000
E = 320000
H = 128
T = 512
G = 256

NC = 2
NS = 16
HH = H // NC
EPT = E // NS
CH = 80
NCH = EPT // CH
NP = 10240
RPT = NP // NS
ZR = 128

def _build_sc_msg():
    mesh = plsc.VectorSubcoreMesh(
        core_axis_name="c", subcore_axis_name="s",
        num_cores=NC, num_subcores=NS,
    )

    @functools.partial(
        pl.kernel,
        out_type=jax.ShapeDtypeStruct((NC, NP, HH), jnp.float32),
        mesh=mesh,
        scratch_types=[
            pltpu.VMEM((N,), jnp.int32),
            pltpu.VMEM((NCH, CH), jnp.int32),
            pltpu.VMEM((NCH, CH), jnp.int32),
            pltpu.VMEM((NCH, CH), jnp.int32),
            pltpu.VMEM((CH, HH), jnp.float32),
            pltpu.VMEM((CH, HH), jnp.float32),
            pltpu.VMEM((ZR, HH), jnp.float32),
            pltpu.VMEM_SHARED((NP, HH), jnp.float32),
            pltpu.SemaphoreType.DMA,
            pltpu.SemaphoreType.DMA,
            pltpu.SemaphoreType.DMA,
            pltpu.SemaphoreType.DMA,
        ],
        compiler_params=pltpu.CompilerParams(
            needs_layout_passes=False, use_tc_tiling_on_sc=False),
    )
    def sc_msg(xs_hbm, src_hbm, dst_hbm, emb_hbm, zeros_hbm, out_hbm,
               xs_v, src_v, dst_v, t_v, rows0, rows1, zbuf, msg_sh,
               gsem0, gsem1, ssem0, ssem1):
        c = lax.axis_index("c")
        s = lax.axis_index("s")

        pltpu.sync_copy(zeros_hbm, zbuf)
        for k in range(RPT // ZR):
            pltpu.sync_copy(zbuf, msg_sh.at[pl.ds(s * RPT + k * ZR, ZR)])
        plsc.subcore_barrier()

        pltpu.sync_copy(xs_hbm, xs_v)
        pltpu.sync_copy(src_hbm.at[s], src_v)
        pltpu.sync_copy(dst_hbm.at[s], dst_v)

        def t_body(r, _):
            for c5 in range(CH // 16):
                src16 = src_v[r, pl.ds(c5 * 16, 16)]
                t16 = plsc.load_gather(xs_v, [src16])
                t_v[r, pl.ds(c5 * 16, 16)] = t16
            return 0

        lax.fori_loop(0, NCH, t_body, 0)

        def g_desc(j, buf, sem):
            return pltpu.make_async_copy(
                emb_hbm.at[c].at[t_v.at[j]], buf, sem)

        def s_desc(j, buf, sem):
            return pltpu.make_async_copy(
                buf, msg_sh.at[dst_v.at[j]], sem)

        def g_start(j, buf, sem):
            pltpu.async_copy(emb_hbm.at[c].at[t_v.at[j]], buf, sem)

        def s_start(j, buf, sem):
            pltpu.async_copy(buf, msg_sh.at[dst_v.at[j]], sem, add=True)

        g_start(0, rows0, gsem0)

        def e_body(jj, _):
            j0 = jj * 2
            j1 = j0 + 1
            g_desc(j0, rows0, gsem0).wait()

            @pl.when(jj > 0)
            def _():
                s_desc(j0 - 1, rows1, ssem1).wait()

            s_start(j0, rows0, ssem0)
            g_start(j1, rows1, gsem1)

            g_desc(j1, rows1, gsem1).wait()
            s_desc(j0, rows0, ssem0).wait()

            @pl.when(j1 + 1 < NCH)
            def _():
                g_start(j1 + 1, rows0, gsem0)

            s_start(j1, rows1, ssem1)
            return 0

        lax.fori_loop(0, NCH // 2, e_body, 0)
        s_desc(NCH - 1, rows1, ssem1).wait()
        plsc.subcore_barrier()

        for k in range(RPT // ZR):
            start = s * RPT + k * ZR
            pltpu.sync_copy(msg_sh.at[pl.ds(start, ZR)], zbuf)
            pltpu.sync_copy(zbuf, out_hbm.at[c].at[pl.ds(start, ZR)])

    return sc_msg


_SC_MSG_CACHE = []


def _sc_msg(*args):
    if not _SC_MSG_CACHE:
        _SC_MSG_CACHE.append(_build_sc_msg())
    return _SC_MSG_CACHE[0](*args)


BLK = 1000
NB = N // BLK
_F32 = jnp.float32
_PREC = lax.Precision.HIGHEST


def _dot_t(a, b):
    return lax.dot_general(a, b, (((1,), (1,)), ((), ())),
                           precision=_PREC, preferred_element_type=_F32)


def _dot(a, b):
    return lax.dot_general(a, b, (((1,), (0,)), ((), ())),
                           precision=_PREC, preferred_element_type=_F32)


def _dot_c0(a, b):
    return lax.dot_general(a, b, (((0,), (0,)), ((), ())),
                           precision=_PREC, preferred_element_type=_F32)


def _tc_body(msg_ref, xs_ref, b_ref, bn_ref, emb_ref, wih_ref, whh_ref,
             w1_ref, w2_ref, b2_ref, wq_ref, bq_ref, wt_ref, out_ref,
             h_sc, wl_sc, wg_sc):
    emb = emb_ref[...]
    wih = wih_ref[...]
    whh = whh_ref[...]
    wl_sc[...] = jnp.zeros((G, H), _F32)
    wg_sc[...] = jnp.zeros((G, H), _F32)

    def pass1(i, _):
        ds = pl.ds(i * BLK, BLK)
        xs = xs_ref[ds, :]
        ohe = (lax.broadcasted_iota(jnp.int32, (BLK, T), 1) == xs
               ).astype(_F32)
        embb = _dot(ohe, emb)
        msgb = jnp.concatenate(
            [msg_ref[0, ds, :], msg_ref[1, ds, :]], axis=1)
        gi = _dot_t(msgb, wih)
        gh = _dot_t(embb, whh)
        r = jax.nn.sigmoid(gi[:, :H] + gh[:, :H])
        z = jax.nn.sigmoid(gi[:, H:2 * H] + gh[:, H:2 * H])
        n = jnp.tanh(gi[:, 2 * H:] + r * gh[:, 2 * H:])
        hb = (1.0 - z) * n + z * embb
        h_sc[ds, :] = hb
        b = b_ref[ds, :]
        ohb = (lax.broadcasted_iota(jnp.int32, (BLK, G), 1) == b
               ).astype(_F32)
        lastm = (b != bn_ref[ds, :]).astype(_F32)
        wl_sc[...] += _dot_c0(ohb * lastm, hb)
        return 0

    lax.fori_loop(0, NB, pass1, 0)
    wl = wl_sc[...]

    def pass2(i, _):
        ds = pl.ds(i * BLK, BLK)
        hb = h_sc[ds, :]
        b = b_ref[ds, :]
        ohb = (lax.broadcasted_iota(jnp.int32, (BLK, G), 1) == b
               ).astype(_F32)
        wgr = _dot(ohb, wl)
        q1 = _dot_t(wgr, w1_ref[...])
        q2 = _dot_t(hb, w2_ref[...]) + b2_ref[...]
        alpha = _dot_t(jax.nn.sigmoid(q1 + q2), wq_ref[...]) + bq_ref[...]
        a = alpha * hb
        wg_sc[...] += _dot_c0(ohb, a)
        return 0

    lax.fori_loop(0, NB, pass2, 0)
    wcat = jnp.concatenate([wl, wg_sc[...]], axis=1)
    wv = _dot_t(wcat, wt_ref[...])
    out_ref[...] = _dot_t(wv, emb)


@jax.jit
def _run(xs, src, dst, batch, bnext, embedding, gru_w_ih, gru_w_hh,
         W1, W2, b2, Wq, bq, Wt):
    zeros = jnp.zeros((ZR, HH), jnp.float32)
    emb2 = jnp.stack([embedding[:, :HH], embedding[:, HH:]])
    msg2 = _sc_msg(xs, src, dst, emb2, zeros)
    return pl.pallas_call(
        _tc_body,
        out_shape=jax.ShapeDtypeStruct((G, T), jnp.float32),
        scratch_shapes=[
            pltpu.VMEM((N, H), jnp.float32),
            pltpu.VMEM((G, H), jnp.float32),
            pltpu.VMEM((G, H), jnp.float32),
        ],
    )(msg2, xs.reshape(N, 1), batch.reshape(N, 1), bnext.reshape(N, 1),
      embedding, gru_w_ih, gru_w_hh, W1, W2, b2.reshape(1, H), Wq,
      bq.reshape(1, H), Wt)


def kernel(x, edge_index, batch, embedding, gru_w_ih, gru_w_hh,
           W1, W2, b2, Wq, bq, Wt):
    xs = x[:, 0].astype(jnp.int32)
    src = edge_index[0].astype(jnp.int32).reshape(NS, NCH, CH)
    dst = edge_index[1].astype(jnp.int32).reshape(NS, NCH, CH)
    bnext = jnp.concatenate(
        [batch[1:], jnp.full((1,), G, batch.dtype)]).astype(jnp.int32)
    return _run(xs, src, dst, batch.astype(jnp.int32), bnext, embedding,
                gru_w_ih, gru_w_hh, W1, W2, b2, Wq, bq, Wt)


# speedup vs baseline: 6.6639x; 1.2946x over previous
import functools

import jax
import jax.numpy as jnp
from jax import lax
from jax.experimental import pallas as pl
from jax.experimental.pallas import tpu as pltpu
from jax.experimental.pallas import tpu_sc as plsc

N = 10000
E = 320000
H = 128
T = 512
G = 256

NC = 2
NS = 16
HH = H // NC
EPT = E // NS
CH = 80
NCH = EPT // CH
NP = 10240
RPT = NP // NS
ZR = 128

def _build_sc_msg():
    mesh = plsc.VectorSubcoreMesh(
        core_axis_name="c", subcore_axis_name="s",
        num_cores=NC, num_subcores=NS,
    )

    @functools.partial(
        pl.kernel,
        out_type=jax.ShapeDtypeStruct((NC, NP, HH), jnp.float32),
        mesh=mesh,
        scratch_types=[
            pltpu.VMEM((N,), jnp.int32),
            pltpu.VMEM((NCH, CH), jnp.int32),
            pltpu.VMEM((NCH, CH), jnp.int32),
            pltpu.VMEM((NCH, CH), jnp.int32),
            pltpu.VMEM((CH, HH), jnp.float32),
            pltpu.VMEM((CH, HH), jnp.float32),
            pltpu.VMEM((ZR, HH), jnp.float32),
            pltpu.VMEM_SHARED((NP, HH), jnp.float32),
            pltpu.SemaphoreType.DMA,
            pltpu.SemaphoreType.DMA,
            pltpu.SemaphoreType.DMA,
            pltpu.SemaphoreType.DMA,
        ],
        compiler_params=pltpu.CompilerParams(
            needs_layout_passes=False, use_tc_tiling_on_sc=False),
    )
    def sc_msg(xs_hbm, src_hbm, dst_hbm, emb_hbm, zeros_hbm, out_hbm,
               xs_v, src_v, dst_v, t_v, rows0, rows1, zbuf, msg_sh,
               gsem0, gsem1, ssem0, ssem1):
        c = lax.axis_index("c")
        s = lax.axis_index("s")

        pltpu.sync_copy(zeros_hbm, zbuf)
        for k in range(RPT // ZR):
            pltpu.sync_copy(zbuf, msg_sh.at[pl.ds(s * RPT + k * ZR, ZR)])
        plsc.subcore_barrier()

        pltpu.sync_copy(xs_hbm, xs_v)
        pltpu.sync_copy(src_hbm.at[s], src_v)
        pltpu.sync_copy(dst_hbm.at[s], dst_v)

        def t_body(r, _):
            for c5 in range(CH // 16):
                src16 = src_v[r, pl.ds(c5 * 16, 16)]
                t16 = plsc.load_gather(xs_v, [src16])
                t_v[r, pl.ds(c5 * 16, 16)] = t16
            return 0

        lax.fori_loop(0, NCH, t_body, 0)

        def g_desc(j, buf, sem):
            return pltpu.make_async_copy(
                emb_hbm.at[c].at[t_v.at[j]], buf, sem)

        def s_desc(j, buf, sem):
            return pltpu.make_async_copy(
                buf, msg_sh.at[dst_v.at[j]], sem)

        def g_start(j, buf, sem):
            pltpu.async_copy(emb_hbm.at[c].at[t_v.at[j]], buf, sem)

        def s_start(j, buf, sem):
            pltpu.async_copy(buf, msg_sh.at[dst_v.at[j]], sem, add=True)

        g_start(0, rows0, gsem0)

        def e_body(jj, _):
            j0 = jj * 2
            j1 = j0 + 1
            g_desc(j0, rows0, gsem0).wait()

            @pl.when(jj > 0)
            def _():
                s_desc(j0 - 1, rows1, ssem1).wait()

            s_start(j0, rows0, ssem0)
            g_start(j1, rows1, gsem1)

            g_desc(j1, rows1, gsem1).wait()
            s_desc(j0, rows0, ssem0).wait()

            @pl.when(j1 + 1 < NCH)
            def _():
                g_start(j1 + 1, rows0, gsem0)

            s_start(j1, rows1, ssem1)
            return 0

        lax.fori_loop(0, NCH // 2, e_body, 0)
        s_desc(NCH - 1, rows1, ssem1).wait()
        plsc.subcore_barrier()

        for k in range(RPT // ZR):
            start = s * RPT + k * ZR
            pltpu.sync_copy(msg_sh.at[pl.ds(start, ZR)], zbuf)
            pltpu.sync_copy(zbuf, out_hbm.at[c].at[pl.ds(start, ZR)])

    return sc_msg


_SC_MSG_CACHE = []


def _sc_msg(*args):
    if not _SC_MSG_CACHE:
        _SC_MSG_CACHE.append(_build_sc_msg())
    return _SC_MSG_CACHE[0](*args)


BLK = 1000
NB = N // BLK
_F32 = jnp.float32
_PREC = lax.Precision.DEFAULT


def _dot_t(a, b):
    return lax.dot_general(a, b, (((1,), (1,)), ((), ())),
                           precision=_PREC, preferred_element_type=_F32)


def _dot(a, b):
    return lax.dot_general(a, b, (((1,), (0,)), ((), ())),
                           precision=_PREC, preferred_element_type=_F32)


def _dot_c0(a, b):
    return lax.dot_general(a, b, (((0,), (0,)), ((), ())),
                           precision=_PREC, preferred_element_type=_F32)


def _tc_body(msg_ref, xs_ref, b_ref, bn_ref, emb_ref, wih_ref, whh_ref,
             w1_ref, w2_ref, b2_ref, wq_ref, bq_ref, wt_ref, out_ref,
             h_sc, wl_sc, wg_sc):
    emb = emb_ref[...]
    wih = wih_ref[...]
    whh = whh_ref[...]
    wl_sc[...] = jnp.zeros((G, H), _F32)
    wg_sc[...] = jnp.zeros((G, H), _F32)

    def pass1(i, _):
        ds = pl.ds(i * BLK, BLK)
        xs = xs_ref[ds, :]
        ohe = (lax.broadcasted_iota(jnp.int32, (BLK, T), 1) == xs
               ).astype(_F32)
        embb = _dot(ohe, emb)
        msgb = jnp.concatenate(
            [msg_ref[0, ds, :], msg_ref[1, ds, :]], axis=1)
        gi = _dot_t(msgb, wih)
        gh = _dot_t(embb, whh)
        r = jax.nn.sigmoid(gi[:, :H] + gh[:, :H])
        z = jax.nn.sigmoid(gi[:, H:2 * H] + gh[:, H:2 * H])
        n = jnp.tanh(gi[:, 2 * H:] + r * gh[:, 2 * H:])
        hb = (1.0 - z) * n + z * embb
        h_sc[ds, :] = hb
        b = b_ref[ds, :]
        ohb = (lax.broadcasted_iota(jnp.int32, (BLK, G), 1) == b
               ).astype(_F32)
        lastm = (b != bn_ref[ds, :]).astype(_F32)
        wl_sc[...] += _dot_c0(ohb * lastm, hb)
        return 0

    lax.fori_loop(0, NB, pass1, 0)
    wl = wl_sc[...]

    def pass2(i, _):
        ds = pl.ds(i * BLK, BLK)
        hb = h_sc[ds, :]
        b = b_ref[ds, :]
        ohb = (lax.broadcasted_iota(jnp.int32, (BLK, G), 1) == b
               ).astype(_F32)
        wgr = _dot(ohb, wl)
        q1 = _dot_t(wgr, w1_ref[...])
        q2 = _dot_t(hb, w2_ref[...]) + b2_ref[...]
        alpha = _dot_t(jax.nn.sigmoid(q1 + q2), wq_ref[...]) + bq_ref[...]
        a = alpha * hb
        wg_sc[...] += _dot_c0(ohb, a)
        return 0

    lax.fori_loop(0, NB, pass2, 0)
    wcat = jnp.concatenate([wl, wg_sc[...]], axis=1)
    wv = _dot_t(wcat, wt_ref[...])
    out_ref[...] = _dot_t(wv, emb)


@jax.jit
def _run(xs, src, dst, batch, bnext, embedding, gru_w_ih, gru_w_hh,
         W1, W2, b2, Wq, bq, Wt):
    zeros = jnp.zeros((ZR, HH), jnp.float32)
    emb2 = jnp.stack([embedding[:, :HH], embedding[:, HH:]])
    msg2 = _sc_msg(xs, src, dst, emb2, zeros)
    return pl.pallas_call(
        _tc_body,
        out_shape=jax.ShapeDtypeStruct((G, T), jnp.float32),
        scratch_shapes=[
            pltpu.VMEM((N, H), jnp.float32),
            pltpu.VMEM((G, H), jnp.float32),
            pltpu.VMEM((G, H), jnp.float32),
        ],
    )(msg2, xs.reshape(N, 1), batch.reshape(N, 1), bnext.reshape(N, 1),
      embedding, gru_w_ih, gru_w_hh, W1, W2, b2.reshape(1, H), Wq,
      bq.reshape(1, H), Wt)


def kernel(x, edge_index, batch, embedding, gru_w_ih, gru_w_hh,
           W1, W2, b2, Wq, bq, Wt):
    xs = x[:, 0].astype(jnp.int32)
    src = edge_index[0].astype(jnp.int32).reshape(NS, NCH, CH)
    dst = edge_index[1].astype(jnp.int32).reshape(NS, NCH, CH)
    bnext = jnp.concatenate(
        [batch[1:], jnp.full((1,), G, batch.dtype)]).astype(jnp.int32)
    return _run(xs, src, dst, batch.astype(jnp.int32), bnext, embedding,
                gru_w_ih, gru_w_hh, W1, W2, b2, Wq, bq, Wt)
